# baseline (speedup 1.0000x reference)
import jax
import jax.numpy as jnp
from jax.experimental import pallas as pl
from jax.experimental.pallas import tpu as pltpu

D_MODEL = 2048
TOK_BLOCK = 512


def _moe_body(idx_ref, x_ref, w0_ref, w1_ref, b0_ref, b1_ref, out_ref):
    x = x_ref[...]
    y0 = jnp.dot(x, w0_ref[...], preferred_element_type=jnp.float32)
    y1 = jnp.dot(x, w1_ref[...], preferred_element_type=jnp.float32)
    y0 = y0 + b0_ref[...]
    y1 = y1 + b1_ref[...]
    mask = idx_ref[...] == 0
    out_ref[...] = jnp.where(mask, y0, y1)


@jax.jit
def _moe(x2d, idxcol, W0T, W1T, b0, b1):
    n_tok = x2d.shape[0]
    grid = (n_tok // TOK_BLOCK,)
    return pl.pallas_call(
        _moe_body,
        grid=grid,
        in_specs=[
            pl.BlockSpec((TOK_BLOCK, 1), lambda i: (i, 0)),
            pl.BlockSpec((TOK_BLOCK, D_MODEL), lambda i: (i, 0)),
            pl.BlockSpec((D_MODEL, D_MODEL), lambda i: (0, 0)),
            pl.BlockSpec((D_MODEL, D_MODEL), lambda i: (0, 0)),
            pl.BlockSpec((1, D_MODEL), lambda i: (0, 0)),
            pl.BlockSpec((1, D_MODEL), lambda i: (0, 0)),
        ],
        out_specs=pl.BlockSpec((TOK_BLOCK, D_MODEL), lambda i: (i, 0)),
        out_shape=jax.ShapeDtypeStruct((n_tok, D_MODEL), jnp.float32),
    )(idxcol, x2d, W0T, W1T, b0, b1)


def kernel(hidden_states, multiway_indices, W0, b0, W1, b1):
    batch, seq, d = hidden_states.shape
    x2d = hidden_states.reshape(batch * seq, d)
    idxcol = multiway_indices.astype(jnp.int32).reshape(batch * seq, 1)
    out = _moe(x2d, idxcol, W0.T, W1.T, b0.reshape(1, d), b1.reshape(1, d))
    return out.reshape(batch, seq, d)


# speedup vs baseline: 1.1215x; 1.1215x over previous
import jax
import jax.numpy as jnp
from jax.experimental import pallas as pl
from jax.experimental.pallas import tpu as pltpu

D_MODEL = 2048
TOK_BLOCK = 512


def _moe_body(idx_ref, x_ref, w0_ref, w1_ref, b0_ref, b1_ref, out_ref):
    x = x_ref[...].astype(jnp.bfloat16)
    y0 = jnp.dot(x, w0_ref[...], preferred_element_type=jnp.float32)
    y1 = jnp.dot(x, w1_ref[...], preferred_element_type=jnp.float32)
    y0 = y0 + b0_ref[...]
    y1 = y1 + b1_ref[...]
    mask = idx_ref[...] == 0
    out_ref[...] = jnp.where(mask, y0, y1)


@jax.jit
def _moe(x2d, idxcol, W0T, W1T, b0, b1):
    n_tok = x2d.shape[0]
    grid = (n_tok // TOK_BLOCK,)
    return pl.pallas_call(
        _moe_body,
        grid=grid,
        in_specs=[
            pl.BlockSpec((TOK_BLOCK, 1), lambda i: (i, 0)),
            pl.BlockSpec((TOK_BLOCK, D_MODEL), lambda i: (i, 0)),
            pl.BlockSpec((D_MODEL, D_MODEL), lambda i: (0, 0)),
            pl.BlockSpec((D_MODEL, D_MODEL), lambda i: (0, 0)),
            pl.BlockSpec((1, D_MODEL), lambda i: (0, 0)),
            pl.BlockSpec((1, D_MODEL), lambda i: (0, 0)),
        ],
        out_specs=pl.BlockSpec((TOK_BLOCK, D_MODEL), lambda i: (i, 0)),
        out_shape=jax.ShapeDtypeStruct((n_tok, D_MODEL), jnp.float32),
        compiler_params=pltpu.CompilerParams(
            dimension_semantics=("parallel",)),
    )(idxcol, x2d, W0T, W1T, b0, b1)


def kernel(hidden_states, multiway_indices, W0, b0, W1, b1):
    batch, seq, d = hidden_states.shape
    x2d = hidden_states.reshape(batch * seq, d)
    idxcol = multiway_indices.astype(jnp.int32).reshape(batch * seq, 1)
    out = _moe(x2d, idxcol, W0.T.astype(jnp.bfloat16), W1.T.astype(jnp.bfloat16),
               b0.reshape(1, d), b1.reshape(1, d))
    return out.reshape(batch, seq, d)
